# baseline (speedup 1.0000x reference)
import functools

import jax
import jax.numpy as jnp
from jax import lax
from jax.experimental import pallas as pl
from jax.experimental.pallas import tpu as pltpu
from jax.experimental.pallas import tpu_sc as plsc


_CHUNK = 16
_NBUF = 7
_LAG = 5

_SP_CHUNK = 128
_SP_NBUF = 8
_SP_LAG = 4

_STREAM_ROWS = 0


def _pipelined_copy(src_hbm, dst_hbm, base, nchunks, chunk, bufs, in_sems,
                    out_sems, lag):
    nbuf = len(bufs)
    in_d = [None] * nchunks
    out_d = [None] * nchunks
    for i in range(nchunks + lag):
        if i < nchunks:
            b = i % nbuf
            if i >= nbuf:
                out_d[i - nbuf].wait()
            in_d[i] = pltpu.async_copy(
                src_hbm.at[pl.ds(base + i * chunk, chunk)], bufs[b],
                in_sems[b])
        if i >= lag:
            j = i - lag
            in_d[j].wait()
            out_d[j] = pltpu.async_copy(
                bufs[j % nbuf], dst_hbm.at[pl.ds(base + j * chunk, chunk)],
                out_sems[j % nbuf])
    for j in range(max(0, nchunks - nbuf), nchunks):
        out_d[j].wait()


@functools.lru_cache(maxsize=None)
def _build_copy(seq_len: int, embed_dim: int, dtype_name: str):
    dtype = jnp.dtype(dtype_name)
    info = plsc.get_sparse_core_info()
    nc, ns = info.num_cores, info.num_subcores
    nw = nc * ns
    stream_rows = _STREAM_ROWS
    sp_rows = seq_len - stream_rows
    assert stream_rows % (nw * _CHUNK) == 0 if stream_rows else True
    assert sp_rows % (nc * _SP_CHUNK) == 0
    rows_per_w = stream_rows // nw
    nchunks = rows_per_w // _CHUNK
    sp_per_sc = sp_rows // nc
    sp_nchunks = sp_per_sc // _SP_CHUNK

    mesh = plsc.VectorSubcoreMesh(core_axis_name="c", subcore_axis_name="s")

    def body(table_hbm, out_hbm, *scratch):
        bufs = scratch[:_NBUF]
        in_sems = scratch[_NBUF:2 * _NBUF]
        out_sems = scratch[2 * _NBUF:3 * _NBUF]
        o = 3 * _NBUF
        sp_bufs = scratch[o:o + _SP_NBUF]
        sp_in_sems = scratch[o + _SP_NBUF:o + 2 * _SP_NBUF]
        sp_out_sems = scratch[o + 2 * _SP_NBUF:o + 3 * _SP_NBUF]
        cid = lax.axis_index("c")
        sid = lax.axis_index("s")

        if nchunks:
            wid = sid * nc + cid
            _pipelined_copy(table_hbm, out_hbm, wid * rows_per_w, nchunks,
                            _CHUNK, bufs, in_sems, out_sems, _LAG)

        if sp_nchunks:
            @pl.when(sid == 0)
            def _():
                _pipelined_copy(table_hbm, out_hbm,
                                stream_rows + cid * sp_per_sc, sp_nchunks,
                                _SP_CHUNK, sp_bufs, sp_in_sems, sp_out_sems,
                                _SP_LAG)

    return pl.kernel(
        body,
        out_type=jax.ShapeDtypeStruct((seq_len, embed_dim), dtype),
        mesh=mesh,
        scratch_types=(
            [pltpu.VMEM((_CHUNK, embed_dim), dtype) for _ in range(_NBUF)]
            + [pltpu.SemaphoreType.DMA for _ in range(2 * _NBUF)]
            + [pltpu.VMEM_SHARED((_SP_CHUNK, embed_dim), dtype)
               for _ in range(_SP_NBUF)]
            + [pltpu.SemaphoreType.DMA for _ in range(2 * _SP_NBUF)]
        ),
    )


def kernel(idx, table):
    seq_len = idx.shape[1]
    return _build_copy(seq_len, table.shape[1], table.dtype.name)(table)


# speedup vs baseline: 1.0355x; 1.0355x over previous
import functools

import jax
import jax.numpy as jnp
from jax import lax
from jax.experimental import pallas as pl
from jax.experimental.pallas import tpu as pltpu
from jax.experimental.pallas import tpu_sc as plsc


_CHUNK = 16
_NBUF = 6
_LAG = 4

_SP_CHUNK = 64
_SP_NBUF = 7
_SP_LAG = 3

_STREAM_ROWS = 3840


def _pipelined_copy(src_hbm, dst_hbm, base, nchunks, chunk, ring, nbuf,
                    in_sems, out_sems, lag):
    in_d = [None] * nchunks
    out_d = [None] * nchunks
    for i in range(nchunks + lag):
        if i < nchunks:
            b = i % nbuf
            if i >= nbuf:
                out_d[i - nbuf].wait()
            in_d[i] = pltpu.async_copy(
                src_hbm.at[pl.ds(base + i * chunk, chunk)], ring.at[b],
                in_sems[b])
        if i >= lag:
            j = i - lag
            in_d[j].wait()
            out_d[j] = pltpu.async_copy(
                ring.at[j % nbuf], dst_hbm.at[pl.ds(base + j * chunk, chunk)],
                out_sems[j % nbuf])
    for j in range(max(0, nchunks - nbuf), nchunks):
        out_d[j].wait()


@functools.lru_cache(maxsize=None)
def _build_copy(seq_len: int, embed_dim: int, dtype_name: str):
    dtype = jnp.dtype(dtype_name)
    info = plsc.get_sparse_core_info()
    nc, ns = info.num_cores, info.num_subcores
    nw = nc * ns
    n_stream_w = nc * (ns - 1)
    stream_rows = _STREAM_ROWS
    sp_rows = seq_len - stream_rows
    if stream_rows:
        assert stream_rows % (n_stream_w * _CHUNK) == 0
    assert sp_rows % (nc * _SP_CHUNK) == 0
    rows_per_w = stream_rows // n_stream_w
    nchunks = rows_per_w // _CHUNK
    sp_per_sc = sp_rows // nc
    sp_nchunks = sp_per_sc // _SP_CHUNK

    mesh = plsc.VectorSubcoreMesh(core_axis_name="c", subcore_axis_name="s")

    def body(table_hbm, out_hbm, ring, sp_ring, *sems):
        in_sems = sems[:_NBUF]
        out_sems = sems[_NBUF:2 * _NBUF]
        o = 2 * _NBUF
        sp_in_sems = sems[o:o + _SP_NBUF]
        sp_out_sems = sems[o + _SP_NBUF:o + 2 * _SP_NBUF]
        cid = lax.axis_index("c")
        sid = lax.axis_index("s")

        if nchunks:
            @pl.when(sid != 0)
            def _():
                wid = (sid - 1) * nc + cid
                _pipelined_copy(table_hbm, out_hbm, wid * rows_per_w,
                                nchunks, _CHUNK, ring, _NBUF, in_sems,
                                out_sems, _LAG)

        if sp_nchunks:
            @pl.when(sid == 0)
            def _():
                _pipelined_copy(table_hbm, out_hbm,
                                stream_rows + cid * sp_per_sc, sp_nchunks,
                                _SP_CHUNK, sp_ring, _SP_NBUF, sp_in_sems,
                                sp_out_sems, _SP_LAG)

    return pl.kernel(
        body,
        out_type=jax.ShapeDtypeStruct((seq_len, embed_dim), dtype),
        mesh=mesh,
        scratch_types=(
            [pltpu.VMEM((_NBUF, _CHUNK, embed_dim), dtype),
             pltpu.VMEM_SHARED((_SP_NBUF, _SP_CHUNK, embed_dim), dtype)]
            + [pltpu.SemaphoreType.DMA for _ in range(2 * _NBUF)]
            + [pltpu.SemaphoreType.DMA for _ in range(2 * _SP_NBUF)]
        ),
    )


def kernel(idx, table):
    seq_len = idx.shape[1]
    return _build_copy(seq_len, table.shape[1], table.dtype.name)(table)


# speedup vs baseline: 1.0576x; 1.0213x over previous
import functools

import jax
import jax.numpy as jnp
from jax import lax
from jax.experimental import pallas as pl
from jax.experimental.pallas import tpu as pltpu
from jax.experimental.pallas import tpu_sc as plsc


_CHUNK = 16
_NBUF = 7
_LAG = 5

_SP_CHUNK = 64
_SP_NBUF = 7
_SP_LAG = 3

_STREAM_ROWS = 8192


def _pipelined_copy(src_hbm, dst_hbm, base, nchunks, chunk, ring, nbuf,
                    in_sems, out_sems, lag):
    in_d = [None] * nchunks
    out_d = [None] * nchunks
    for i in range(nchunks + lag):
        if i < nchunks:
            b = i % nbuf
            if i >= nbuf:
                out_d[i - nbuf].wait()
            in_d[i] = pltpu.async_copy(
                src_hbm.at[pl.ds(base + i * chunk, chunk)], ring.at[b],
                in_sems[b])
        if i >= lag:
            j = i - lag
            in_d[j].wait()
            out_d[j] = pltpu.async_copy(
                ring.at[j % nbuf], dst_hbm.at[pl.ds(base + j * chunk, chunk)],
                out_sems[j % nbuf])
    for j in range(max(0, nchunks - nbuf), nchunks):
        out_d[j].wait()


@functools.lru_cache(maxsize=None)
def _build_copy(seq_len: int, embed_dim: int, dtype_name: str):
    dtype = jnp.dtype(dtype_name)
    info = plsc.get_sparse_core_info()
    nc, ns = info.num_cores, info.num_subcores
    stream_rows = _STREAM_ROWS
    sp_rows = seq_len - stream_rows
    use_sp = sp_rows > 0
    n_stream_w = nc * (ns - 1) if use_sp else nc * ns
    if stream_rows:
        assert stream_rows % (n_stream_w * _CHUNK) == 0
    assert sp_rows % (nc * _SP_CHUNK) == 0
    rows_per_w = stream_rows // n_stream_w
    nchunks = rows_per_w // _CHUNK
    sp_per_sc = sp_rows // nc
    sp_nchunks = sp_per_sc // _SP_CHUNK

    mesh = plsc.VectorSubcoreMesh(core_axis_name="c", subcore_axis_name="s")

    def body(table_hbm, out_hbm, *rest):
        ring = rest[0]
        sems = rest[2:] if use_sp else rest[1:]
        in_sems = sems[:_NBUF]
        out_sems = sems[_NBUF:2 * _NBUF]
        cid = lax.axis_index("c")
        sid = lax.axis_index("s")

        if nchunks and use_sp:
            @pl.when(sid != 0)
            def _():
                wid = (sid - 1) * nc + cid
                _pipelined_copy(table_hbm, out_hbm, wid * rows_per_w,
                                nchunks, _CHUNK, ring, _NBUF, in_sems,
                                out_sems, _LAG)
        elif nchunks:
            wid = sid * nc + cid
            _pipelined_copy(table_hbm, out_hbm, wid * rows_per_w, nchunks,
                            _CHUNK, ring, _NBUF, in_sems, out_sems, _LAG)

        if use_sp:
            sp_ring = rest[1]
            o = 2 * _NBUF
            sp_in_sems = sems[o:o + _SP_NBUF]
            sp_out_sems = sems[o + _SP_NBUF:o + 2 * _SP_NBUF]

            @pl.when(sid == 0)
            def _():
                _pipelined_copy(table_hbm, out_hbm,
                                stream_rows + cid * sp_per_sc, sp_nchunks,
                                _SP_CHUNK, sp_ring, _SP_NBUF, sp_in_sems,
                                sp_out_sems, _SP_LAG)

    scratch = [pltpu.VMEM((_NBUF, _CHUNK, embed_dim), dtype)]
    if use_sp:
        scratch.append(
            pltpu.VMEM_SHARED((_SP_NBUF, _SP_CHUNK, embed_dim), dtype))
    scratch += [pltpu.SemaphoreType.DMA for _ in range(2 * _NBUF)]
    if use_sp:
        scratch += [pltpu.SemaphoreType.DMA for _ in range(2 * _SP_NBUF)]

    return pl.kernel(
        body,
        out_type=jax.ShapeDtypeStruct((seq_len, embed_dim), dtype),
        mesh=mesh,
        scratch_types=scratch,
    )


def kernel(idx, table):
    seq_len = idx.shape[1]
    return _build_copy(seq_len, table.shape[1], table.dtype.name)(table)


# speedup vs baseline: 1.0618x; 1.0039x over previous
import functools

import jax
import jax.numpy as jnp
from jax import lax
from jax.experimental import pallas as pl
from jax.experimental.pallas import tpu as pltpu
from jax.experimental.pallas import tpu_sc as plsc


_CHUNK = 16
_NBUF = 7
_LAG = 6

_SP_CHUNK = 64
_SP_NBUF = 7
_SP_LAG = 3

_STREAM_ROWS = 8192


def _pipelined_copy(src_hbm, dst_hbm, base, nchunks, chunk, ring, nbuf,
                    in_sems, out_sems, lag):
    in_d = [None] * nchunks
    out_d = [None] * nchunks
    for i in range(nchunks + lag):
        if i < nchunks:
            b = i % nbuf
            if i >= nbuf:
                out_d[i - nbuf].wait()
            in_d[i] = pltpu.async_copy(
                src_hbm.at[pl.ds(base + i * chunk, chunk)], ring.at[b],
                in_sems[b])
        if i >= lag:
            j = i - lag
            in_d[j].wait()
            out_d[j] = pltpu.async_copy(
                ring.at[j % nbuf], dst_hbm.at[pl.ds(base + j * chunk, chunk)],
                out_sems[j % nbuf])
    for j in range(max(0, nchunks - nbuf), nchunks):
        out_d[j].wait()


@functools.lru_cache(maxsize=None)
def _build_copy(seq_len: int, embed_dim: int, dtype_name: str):
    dtype = jnp.dtype(dtype_name)
    info = plsc.get_sparse_core_info()
    nc, ns = info.num_cores, info.num_subcores
    stream_rows = _STREAM_ROWS
    sp_rows = seq_len - stream_rows
    use_sp = sp_rows > 0
    n_stream_w = nc * (ns - 1) if use_sp else nc * ns
    if stream_rows:
        assert stream_rows % (n_stream_w * _CHUNK) == 0
    assert sp_rows % (nc * _SP_CHUNK) == 0
    rows_per_w = stream_rows // n_stream_w
    nchunks = rows_per_w // _CHUNK
    sp_per_sc = sp_rows // nc
    sp_nchunks = sp_per_sc // _SP_CHUNK

    mesh = plsc.VectorSubcoreMesh(core_axis_name="c", subcore_axis_name="s")

    def body(table_hbm, out_hbm, *rest):
        ring = rest[0]
        sems = rest[2:] if use_sp else rest[1:]
        in_sems = sems[:_NBUF]
        out_sems = sems[_NBUF:2 * _NBUF]
        cid = lax.axis_index("c")
        sid = lax.axis_index("s")

        if nchunks and use_sp:
            @pl.when(sid != 0)
            def _():
                wid = (sid - 1) * nc + cid
                _pipelined_copy(table_hbm, out_hbm, wid * rows_per_w,
                                nchunks, _CHUNK, ring, _NBUF, in_sems,
                                out_sems, _LAG)
        elif nchunks:
            wid = sid * nc + cid
            _pipelined_copy(table_hbm, out_hbm, wid * rows_per_w, nchunks,
                            _CHUNK, ring, _NBUF, in_sems, out_sems, _LAG)

        if use_sp:
            sp_ring = rest[1]
            o = 2 * _NBUF
            sp_in_sems = sems[o:o + _SP_NBUF]
            sp_out_sems = sems[o + _SP_NBUF:o + 2 * _SP_NBUF]

            @pl.when(sid == 0)
            def _():
                _pipelined_copy(table_hbm, out_hbm,
                                stream_rows + cid * sp_per_sc, sp_nchunks,
                                _SP_CHUNK, sp_ring, _SP_NBUF, sp_in_sems,
                                sp_out_sems, _SP_LAG)

    scratch = [pltpu.VMEM((_NBUF, _CHUNK, embed_dim), dtype)]
    if use_sp:
        scratch.append(
            pltpu.VMEM_SHARED((_SP_NBUF, _SP_CHUNK, embed_dim), dtype))
    scratch += [pltpu.SemaphoreType.DMA for _ in range(2 * _NBUF)]
    if use_sp:
        scratch += [pltpu.SemaphoreType.DMA for _ in range(2 * _SP_NBUF)]

    return pl.kernel(
        body,
        out_type=jax.ShapeDtypeStruct((seq_len, embed_dim), dtype),
        mesh=mesh,
        scratch_types=scratch,
    )


def kernel(idx, table):
    seq_len = idx.shape[1]
    return _build_copy(seq_len, table.shape[1], table.dtype.name)(table)
